# baseline (speedup 1.0000x reference)
import functools

import jax
import jax.numpy as jnp
from jax import lax
from jax.experimental import pallas as pl
from jax.experimental.pallas import tpu as pltpu
from jax.experimental.pallas import tpu_sc as plsc

_B, _S = 4, 4096
_D = 128
_N = _B * _S
_CHUNK = 128
_NROWS = _N // _CHUNK


_NBUF = 3
_LOOKAHEAD = 2


@functools.cache
def _build_gather():
    mesh = plsc.VectorSubcoreMesh(core_axis_name="c", subcore_axis_name="s")
    nw = mesh.num_cores * mesh.num_subcores
    rows_per_w = _NROWS // nw
    pairs = rows_per_w

    @functools.partial(
        pl.kernel,
        out_type=(
            jax.ShapeDtypeStruct((_N, _D), jnp.float32),
            jax.ShapeDtypeStruct((_N, _D), jnp.float32),
        ),
        mesh=mesh,
        scratch_types=[
            pltpu.VMEM((rows_per_w, _CHUNK), jnp.int32),
            pltpu.VMEM((_NBUF, 2 * _CHUNK, _D), jnp.float32),
            pltpu.SemaphoreType.DMA((_NBUF,)),
            pltpu.SemaphoreType.DMA((_NBUF,)),
        ],
    )
    def gather_kernel(cos_hbm, sin_hbm, idx_hbm, cos_out, sin_out,
                      idx_v, bufs, gsem, osem):
        wid = lax.axis_index("s") * mesh.num_cores + lax.axis_index("c")
        base_row = wid * rows_per_w
        pltpu.sync_copy(idx_hbm.at[pl.ds(base_row, rows_per_w)], idx_v)

        def tab_out(t):
            return (cos_hbm, cos_out) if t < rows_per_w else (sin_hbm, sin_out)

        g, o = {}, {}

        def fire_out(p):
            b = p % _NBUF
            g[2 * p].wait()
            g[2 * p + 1].wait()
            _, out = tab_out(2 * p)
            o[p] = pltpu.async_copy(
                bufs.at[b],
                out.at[pl.ds((base_row + (2 * p) % rows_per_w) * _CHUNK,
                             2 * _CHUNK)],
                osem.at[b])

        for p in range(pairs):
            b = p % _NBUF
            if p >= _NBUF:
                o[p - _NBUF].wait()
            for h in range(2):
                t = 2 * p + h
                tab, _ = tab_out(t)
                g[t] = pltpu.async_copy(
                    tab.at[idx_v.at[t % rows_per_w]],
                    bufs.at[b].at[pl.ds(h * _CHUNK, _CHUNK)],
                    gsem.at[b])
            if p >= _LOOKAHEAD:
                fire_out(p - _LOOKAHEAD)
        for p in range(pairs - _LOOKAHEAD, pairs):
            fire_out(p)
        for p in range(max(0, pairs - _NBUF), pairs):
            o[p].wait()

    return gather_kernel


def kernel(x, position_ids, cos_cached, sin_cached):
    idx = position_ids.reshape(_NROWS, _CHUNK)
    cos_out, sin_out = _build_gather()(cos_cached, sin_cached, idx)
    return (cos_out.reshape(_B, _S, _D), sin_out.reshape(_B, _S, _D))


# speedup vs baseline: 1.0462x; 1.0462x over previous
import functools

import jax
import jax.numpy as jnp
from jax import lax
from jax.experimental import pallas as pl
from jax.experimental.pallas import tpu as pltpu
from jax.experimental.pallas import tpu_sc as plsc

_B, _S = 4, 4096
_D = 128
_N = _B * _S
_CHUNK = 128
_NROWS = _N // _CHUNK


_NBUF = 6
_LOOKAHEAD = 4


@functools.cache
def _build_gather():
    mesh = plsc.VectorSubcoreMesh(core_axis_name="c", subcore_axis_name="s")
    nw = mesh.num_cores * mesh.num_subcores
    rows_per_w = _NROWS // nw
    steps = 2 * rows_per_w

    @functools.partial(
        pl.kernel,
        out_type=(
            jax.ShapeDtypeStruct((_N, _D), jnp.float32),
            jax.ShapeDtypeStruct((_N, _D), jnp.float32),
        ),
        mesh=mesh,
        scratch_types=[
            pltpu.VMEM((rows_per_w, _CHUNK), jnp.int32),
            pltpu.VMEM((_NBUF, _CHUNK, _D), jnp.float32),
            pltpu.SemaphoreType.DMA((_NBUF,)),
            pltpu.SemaphoreType.DMA((_NBUF,)),
        ],
    )
    def gather_kernel(cos_hbm, sin_hbm, idx_hbm, cos_out, sin_out,
                      idx_v, bufs, gsem, osem):
        wid = lax.axis_index("s") * mesh.num_cores + lax.axis_index("c")
        base_row = wid * rows_per_w
        pltpu.sync_copy(idx_hbm.at[pl.ds(base_row, rows_per_w)], idx_v)

        def tab_out(s):
            j = s // 2
            if s % 2 == 0:
                return cos_hbm, cos_out, j
            return sin_hbm, sin_out, j

        g, o = {}, {}

        def fire_out(t):
            b = t % _NBUF
            g[t].wait()
            _, out, j = tab_out(t)
            o[t] = pltpu.async_copy(
                bufs.at[b],
                out.at[pl.ds((base_row + j) * _CHUNK, _CHUNK)],
                osem.at[b])

        for s in range(steps):
            b = s % _NBUF
            if s >= _NBUF:
                o[s - _NBUF].wait()
            tab, _, j = tab_out(s)
            g[s] = pltpu.async_copy(
                tab.at[idx_v.at[j]], bufs.at[b], gsem.at[b])
            if s >= _LOOKAHEAD:
                fire_out(s - _LOOKAHEAD)
        for t in range(steps - _LOOKAHEAD, steps):
            fire_out(t)
        for t in range(max(0, steps - _NBUF), steps):
            o[t].wait()

    return gather_kernel


def kernel(x, position_ids, cos_cached, sin_cached):
    idx = position_ids.reshape(_NROWS, _CHUNK)
    cos_out, sin_out = _build_gather()(cos_cached, sin_cached, idx)
    return (cos_out.reshape(_B, _S, _D), sin_out.reshape(_B, _S, _D))
